# baseline (speedup 1.0000x reference)
import math

import jax
import jax.numpy as jnp
import numpy as np
from jax.experimental import pallas as pl

_HALF = 32
_DIM = 64
_MAX_PERIOD = 10000.0


def _encode_body(x_ref, m_ref, p_ref, o_ref):
    args = jnp.dot(x_ref[...], m_ref[...], preferred_element_type=jnp.float32)
    o_ref[...] = jnp.sin(args + p_ref[...])


def kernel(data):
    n_rows, n_feat = data.shape
    out_dim = n_feat * _DIM

    freqs = np.exp(
        -math.log(_MAX_PERIOD) * np.arange(_HALF, dtype=np.float32) / _HALF
    ).astype(np.float32)
    tab = np.concatenate([freqs, freqs])
    m = np.zeros((n_feat, out_dim), dtype=np.float32)
    for f in range(n_feat):
        m[f, f * _DIM : (f + 1) * _DIM] = tab
    phase_block = np.zeros((_DIM,), dtype=np.float32)
    phase_block[:_HALF] = np.pi / 2.0
    phase = np.tile(phase_block, n_feat)[None, :]

    blk = 1024
    grid = (n_rows // blk,)

    return pl.pallas_call(
        _encode_body,
        grid=grid,
        in_specs=[
            pl.BlockSpec((blk, n_feat), lambda i: (i, 0)),
            pl.BlockSpec((n_feat, out_dim), lambda i: (0, 0)),
            pl.BlockSpec((1, out_dim), lambda i: (0, 0)),
        ],
        out_specs=pl.BlockSpec((blk, out_dim), lambda i: (i, 0)),
        out_shape=jax.ShapeDtypeStruct((n_rows, out_dim), jnp.float32),
    )(data, jnp.asarray(m), jnp.asarray(phase))
